# baseline (speedup 1.0000x reference)
import jax
import jax.numpy as jnp
from jax.experimental import pallas as pl

_FG = 1
_POS_IOU = 0.2
_ENT_THR = 0.1
_RM_THR = 0.9
_THETA = 0.1

_INTERPRET = False


def _max_body(x_ref, o_ref):
    o_ref[...] = jnp.max(x_ref[...], axis=-1, keepdims=True)[None]


def _loss_body(gmat_ref, pjg_ref, gjg_ref, target_row_ref, gts_row_ref,
               gts_col_ref, u_ref, iou_out_ref, cls_out_ref):
    n = pjg_ref.shape[0]
    m = gmat_ref.shape[0]
    c = gmat_ref.shape[1] - 2

    lane = jax.lax.broadcasted_iota(jnp.int32, (n, m), 1)
    p_onehot = (pjg_ref[...] == lane).astype(jnp.float32)
    g = jnp.dot(p_onehot, gmat_ref[...],
                preferred_element_type=jnp.float32)
    mp = g[:, 0:1]
    preds_iou = g[:, 1:2]
    logits = g[:, 2:]

    t_onehot = (gjg_ref[...] == lane).astype(jnp.float32)
    tid = jnp.sum(t_onehot * target_row_ref[...], axis=1, keepdims=True)
    cls = tid.astype(jnp.int32)

    removed = (mp < _ENT_THR) & (u_ref[...] < _RM_THR)
    gts_iou_col = gts_col_ref[...]
    w = jnp.where(removed, 0.0,
                  jnp.where(gts_iou_col < _POS_IOU, 1.0, 2.0))
    wsum = jnp.sum(w) + 0.0001

    mx = jnp.max(logits, axis=1, keepdims=True)
    lse = mx + jnp.log(jnp.sum(jnp.exp(logits - mx), axis=1, keepdims=True))
    lane_c = jax.lax.broadcasted_iota(jnp.int32, (n, c), 1)
    picked = jnp.sum(jnp.where(lane_c == cls, logits, 0.0), axis=1,
                     keepdims=True)
    cls_l = lse - picked
    cls_loss = jnp.sum(cls_l * w) / wsum

    d = jnp.abs(preds_iou - gts_row_ref[...])
    f = jnp.where(d < _THETA, d * d * (1.0 / (2.0 * _THETA)),
                  d - 0.5 * _THETA)
    iou_num = jnp.sum(jnp.dot(f, w, preferred_element_type=jnp.float32))
    iou_loss = iou_num / wsum

    iou_out_ref[...] = jnp.reshape(iou_loss, (1, 1))
    cls_out_ref[...] = jnp.reshape(cls_loss, (1, 1))


@jax.jit
def kernel(cls_logits, iou_scores, map_ious, pred_mask_prob, target_ids,
           map_indices):
    bs, ch, c = cls_logits.shape
    hw = pred_mask_prob.shape[2] * pred_mask_prob.shape[3]
    rows = bs * ch

    pm = pred_mask_prob.reshape(rows, hw)
    r_blk = 16
    nblk = rows // r_blk
    mx = pl.pallas_call(
        _max_body,
        grid=(nblk,),
        in_specs=[pl.BlockSpec((r_blk, hw), lambda i: (i, 0))],
        out_specs=pl.BlockSpec((1, r_blk, 1), lambda i: (i, 0, 0)),
        out_shape=jax.ShapeDtypeStruct((nblk, r_blk, 1), jnp.float32),
        interpret=_INTERPRET,
    )(pm)
    maxprob = mx.reshape(rows)

    pj = map_indices[:, 0, _FG:].astype(jnp.int32)
    gj = map_indices[:, 1, _FG:].astype(jnp.int32)
    off = (jnp.arange(bs, dtype=jnp.int32) * ch)[:, None]
    pjg = (pj + off).reshape(-1, 1)
    gjg = (gj + off).reshape(-1, 1)
    gmat = jnp.concatenate(
        [maxprob[:, None], iou_scores.reshape(rows, 1),
         cls_logits.reshape(rows, c)], axis=1)
    target_row = target_ids.astype(jnp.float32).reshape(1, rows)
    iou = map_ious[:, _FG:].astype(jnp.float32)
    gts_col = iou.reshape(-1, 1)
    gts_row = iou.reshape(1, -1)
    u = jax.random.uniform(jax.random.key(42), pj.shape, dtype=jnp.float32)
    u_col = u.reshape(-1, 1)

    iou_loss, cls_loss = pl.pallas_call(
        _loss_body,
        out_shape=[jax.ShapeDtypeStruct((1, 1), jnp.float32),
                   jax.ShapeDtypeStruct((1, 1), jnp.float32)],
        interpret=_INTERPRET,
    )(gmat, pjg, gjg, target_row, gts_row, gts_col, u_col)
    return (iou_loss[0, 0], cls_loss[0, 0])


# speedup vs baseline: 1.9639x; 1.9639x over previous
import jax
import jax.numpy as jnp
from jax.experimental import pallas as pl

_FG = 1
_POS_IOU = 0.2
_ENT_THR = 0.1
_RM_THR = 0.9
_THETA = 0.1

_INTERPRET = False


def _max_body(x_ref, o_ref):
    o_ref[...] = jnp.max(x_ref[...], axis=(-2, -1))[None, None]


def _loss_body(gmat_ref, pjg_ref, gjg_ref, target_row_ref, gts_row_ref,
               gts_col_ref, u_ref, iou_out_ref, cls_out_ref):
    n = pjg_ref.shape[0]
    m = gmat_ref.shape[0]
    c = gmat_ref.shape[1] - 2

    lane = jax.lax.broadcasted_iota(jnp.int32, (n, m), 1)
    p_onehot = (pjg_ref[...] == lane).astype(jnp.float32)
    g = jnp.dot(p_onehot, gmat_ref[...],
                preferred_element_type=jnp.float32)
    mp = g[:, 0:1]
    preds_iou = g[:, 1:2]
    logits = g[:, 2:]

    t_onehot = (gjg_ref[...] == lane).astype(jnp.float32)
    tid = jnp.sum(t_onehot * target_row_ref[...], axis=1, keepdims=True)
    cls = tid.astype(jnp.int32)

    removed = (mp < _ENT_THR) & (u_ref[...] < _RM_THR)
    gts_iou_col = gts_col_ref[...]
    w = jnp.where(removed, 0.0,
                  jnp.where(gts_iou_col < _POS_IOU, 1.0, 2.0))
    wsum = jnp.sum(w) + 0.0001

    mx = jnp.max(logits, axis=1, keepdims=True)
    lse = mx + jnp.log(jnp.sum(jnp.exp(logits - mx), axis=1, keepdims=True))
    lane_c = jax.lax.broadcasted_iota(jnp.int32, (n, c), 1)
    picked = jnp.sum(jnp.where(lane_c == cls, logits, 0.0), axis=1,
                     keepdims=True)
    cls_l = lse - picked
    cls_loss = jnp.sum(cls_l * w) / wsum

    d = jnp.abs(preds_iou - gts_row_ref[...])
    f = jnp.where(d < _THETA, d * d * (1.0 / (2.0 * _THETA)),
                  d - 0.5 * _THETA)
    iou_num = jnp.sum(jnp.dot(f, w, preferred_element_type=jnp.float32))
    iou_loss = iou_num / wsum

    iou_out_ref[...] = jnp.reshape(iou_loss, (1, 1))
    cls_out_ref[...] = jnp.reshape(cls_loss, (1, 1))


@jax.jit
def kernel(cls_logits, iou_scores, map_ious, pred_mask_prob, target_ids,
           map_indices):
    bs, ch, c = cls_logits.shape
    hw = pred_mask_prob.shape[2] * pred_mask_prob.shape[3]
    rows = bs * ch

    ht, wd = pred_mask_prob.shape[2], pred_mask_prob.shape[3]
    cc = 20
    nc = ch // cc
    mx = pl.pallas_call(
        _max_body,
        grid=(bs, nc),
        in_specs=[pl.BlockSpec((1, cc, ht, wd), lambda b, c: (b, c, 0, 0))],
        out_specs=pl.BlockSpec((1, 1, 1, cc), lambda b, c: (b, c, 0, 0)),
        out_shape=jax.ShapeDtypeStruct((bs, nc, 1, cc), jnp.float32),
        interpret=_INTERPRET,
    )(pred_mask_prob)
    maxprob = mx.reshape(rows)

    pj = map_indices[:, 0, _FG:].astype(jnp.int32)
    gj = map_indices[:, 1, _FG:].astype(jnp.int32)
    off = (jnp.arange(bs, dtype=jnp.int32) * ch)[:, None]
    pjg = (pj + off).reshape(-1, 1)
    gjg = (gj + off).reshape(-1, 1)
    gmat = jnp.concatenate(
        [maxprob[:, None], iou_scores.reshape(rows, 1),
         cls_logits.reshape(rows, c)], axis=1)
    target_row = target_ids.astype(jnp.float32).reshape(1, rows)
    iou = map_ious[:, _FG:].astype(jnp.float32)
    gts_col = iou.reshape(-1, 1)
    gts_row = iou.reshape(1, -1)
    u = jax.random.uniform(jax.random.key(42), pj.shape, dtype=jnp.float32)
    u_col = u.reshape(-1, 1)

    iou_loss, cls_loss = pl.pallas_call(
        _loss_body,
        out_shape=[jax.ShapeDtypeStruct((1, 1), jnp.float32),
                   jax.ShapeDtypeStruct((1, 1), jnp.float32)],
        interpret=_INTERPRET,
    )(gmat, pjg, gjg, target_row, gts_row, gts_col, u_col)
    return (iou_loss[0, 0], cls_loss[0, 0])
